# Initial kernel scaffold; baseline (speedup 1.0000x reference)
import jax
import jax.numpy as jnp
from jax.experimental import pallas as pl


def kernel(x_tokens, word_type, edge_index, edge_attr, mask_pad, mask_score, emb_table, wt_table, W_w, b_w, gcn1_W, gcn1_b, gcn2_W, gcn2_b, pool1_W, pool1_b, pool2_W, lx_Wih, lx_Whh, lx_bih, lx_bhh, lg_Wih, lg_Whh, lg_bih, lg_bhh):
    raise NotImplementedError("write your pallas kernel here")


# speedup vs baseline: 9.7133x; 9.7133x over previous
import functools

import jax
import jax.numpy as jnp
from jax import lax
from jax.experimental import pallas as pl
from jax.experimental.pallas import tpu as pltpu
from jax.experimental.pallas import tpu_sc as plsc

_N = 10000
_E = 320000
_B = 50
_L = 200
_D = 128

_NC = 2
_NS = 16
_NW = _NC * _NS

_RC = 80
_NCH = _N // _RC
_PE = _E // _NW
_DC = 2000
_EC = 80

_f32 = jnp.float32
_i32 = jnp.int32


def _scale_rows(buf, sbuf, n):
    def body(g, _):
        sv = sbuf[pl.ds(g * 16, 16)]
        for l in range(16):
            i = g * 16 + l
            s = sv[l]
            for j in range(_D // 16):
                sl = pl.ds(j * 16, 16)
                buf[i, sl] = buf[i, sl] * s
        return 0
    lax.fori_loop(0, n // 16, body, 0)


def _zero_rows(buf, n):
    z = jnp.zeros((16,), _f32)
    def body(i, _):
        for j in range(_D // 16):
            buf[i, pl.ds(j * 16, 16)] = z
        return 0
    lax.fori_loop(0, n, body, 0)


def _sc_prep(tok, wty, col, ew, emb_table, wt_table):
    mesh = plsc.VectorSubcoreMesh(core_axis_name="c", subcore_axis_name="s")

    @functools.partial(
        pl.kernel,
        mesh=mesh,
        out_type=(
            jax.ShapeDtypeStruct((_N, _D), _f32),
            jax.ShapeDtypeStruct((2 * _N,), _f32),
        ),
        scratch_types=[
            pltpu.VMEM((_RC,), _i32),
            pltpu.VMEM((_RC,), _i32),
            pltpu.VMEM((_RC, _D), _f32),
            pltpu.VMEM((_RC, _D), _f32),
            pltpu.VMEM((_DC,), _i32),
            pltpu.VMEM((_DC,), _f32),
            pltpu.VMEM_SHARED((_N,), _f32),
            pltpu.SemaphoreType.DMA,
            pltpu.SemaphoreType.DMA,
        ],
    )
    def k(tok_h, wty_h, col_h, ew_h, emb_h, wt_h, embx_o, deg_o,
          tokbuf, wtbuf, ebuf, wbuf, colbuf, ewbuf, deg_acc, sem1, sem2):
        cid = lax.axis_index("c")
        sid = lax.axis_index("s")
        w = sid * _NC + cid

        def zb(i, _):
            ewbuf[pl.ds(i * 16, 16)] = jnp.zeros((16,), _f32)
            return 0
        lax.fori_loop(0, _DC // 16, zb, 0)

        @pl.when(sid < _N // _DC)
        def _():
            pltpu.sync_copy(ewbuf, deg_acc.at[pl.ds(sid * _DC, _DC)])
        plsc.subcore_barrier()

        def dchunk(kk, _):
            base = w * _PE + kk * _DC
            pltpu.sync_copy(col_h.at[pl.ds(base, _DC)], colbuf)
            pltpu.sync_copy(ew_h.at[pl.ds(base, _DC)], ewbuf)
            pltpu.sync_copy(ewbuf, deg_acc.at[colbuf], add=True)
            return 0
        lax.fori_loop(0, _PE // _DC, dchunk, 0)
        plsc.subcore_barrier()

        @pl.when(sid < _N // _DC)
        def _():
            pltpu.sync_copy(deg_acc.at[pl.ds(sid * _DC, _DC)], ewbuf)
            pltpu.sync_copy(ewbuf, deg_o.at[pl.ds(cid * _N + sid * _DC, _DC)])

        for r in range((_NCH + _NW - 1) // _NW):
            ck = w + _NW * r

            @pl.when(ck < _NCH)
            def _():
                base = ck * _RC
                pltpu.sync_copy(tok_h.at[pl.ds(base, _RC)], tokbuf)
                pltpu.sync_copy(wty_h.at[pl.ds(base, _RC)], wtbuf)
                cp1 = pltpu.async_copy(emb_h.at[tokbuf], ebuf, sem1)
                cp2 = pltpu.async_copy(wt_h.at[wtbuf], wbuf, sem2)
                cp1.wait()
                cp2.wait()

                def addb(i, _):
                    for j in range(_D // 16):
                        sl = pl.ds(j * 16, 16)
                        ebuf[i, sl] = ebuf[i, sl] + wbuf[i, sl]
                    return 0
                lax.fori_loop(0, _RC, addb, 0)
                pltpu.sync_copy(ebuf, embx_o.at[pl.ds(base, _RC)])

    return k(tok, wty, col, ew, emb_table, wt_table)


def _sc_conv(h, dis, row, col, ew):
    mesh = plsc.VectorSubcoreMesh(core_axis_name="c", subcore_axis_name="s")

    @functools.partial(
        pl.kernel,
        mesh=mesh,
        out_type=jax.ShapeDtypeStruct((2 * _N, _D), _f32),
        scratch_types=[
            pltpu.VMEM((_EC,), _i32),
            pltpu.VMEM((_EC,), _i32),
            pltpu.VMEM((_EC,), _f32),
            pltpu.VMEM((_EC,), _f32),
            pltpu.VMEM((_EC, _D), _f32),
            pltpu.VMEM((_RC, _D), _f32),
            pltpu.VMEM((_RC,), _f32),
            pltpu.VMEM_SHARED((_N, _D), _f32),
            pltpu.SemaphoreType.DMA,
            pltpu.SemaphoreType.DMA,
        ],
    )
    def k(h_h, dis_h, row_h, col_h, ew_h, out_o,
          rowbuf, colbuf, ewbuf, dgbuf, rowsbuf, ibuf, idbuf, acc, sem1, sem2):
        cid = lax.axis_index("c")
        sid = lax.axis_index("s")
        w = sid * _NC + cid

        @pl.when(cid != 0)
        def _():
            _zero_rows(ibuf, _RC)

        for r in range((_NCH + _NS - 1) // _NS):
            ck = sid + _NS * r

            @pl.when(ck < _NCH)
            def _():
                base = ck * _RC

                @pl.when(cid == 0)
                def _():
                    pltpu.sync_copy(h_h.at[pl.ds(base, _RC)], ibuf)
                    pltpu.sync_copy(dis_h.at[pl.ds(base, _RC)], idbuf)
                    _scale_rows(ibuf, idbuf, _RC)
                pltpu.sync_copy(ibuf, acc.at[pl.ds(base, _RC)])
        plsc.subcore_barrier()

        def echunk(kk, _):
            base = w * _PE + kk * _EC
            pltpu.sync_copy(row_h.at[pl.ds(base, _EC)], rowbuf)
            pltpu.sync_copy(ew_h.at[pl.ds(base, _EC)], ewbuf)
            pltpu.sync_copy(col_h.at[pl.ds(base, _EC)], colbuf)
            g1 = pltpu.async_copy(h_h.at[rowbuf], rowsbuf, sem1)
            g2 = pltpu.async_copy(dis_h.at[rowbuf], dgbuf, sem2)
            g1.wait()
            g2.wait()

            def sb(g, _):
                sv = ewbuf[pl.ds(g * 16, 16)] * dgbuf[pl.ds(g * 16, 16)]
                for l in range(16):
                    i = g * 16 + l
                    s = sv[l]
                    for j in range(_D // 16):
                        sl = pl.ds(j * 16, 16)
                        rowsbuf[i, sl] = rowsbuf[i, sl] * s
                return 0
            lax.fori_loop(0, _EC // 16, sb, 0)
            pltpu.sync_copy(rowsbuf, acc.at[colbuf], add=True)
            return 0
        lax.fori_loop(0, _PE // _EC, echunk, 0)
        plsc.subcore_barrier()

        for r in range((_NCH + _NS - 1) // _NS):
            ck = sid + _NS * r

            @pl.when(ck < _NCH)
            def _():
                base = ck * _RC
                pltpu.sync_copy(acc.at[pl.ds(base, _RC)], ibuf)
                pltpu.sync_copy(dis_h.at[pl.ds(base, _RC)], idbuf)
                _scale_rows(ibuf, idbuf, _RC)
                pltpu.sync_copy(ibuf, out_o.at[pl.ds(cid * _N + base, _RC)])

    return k(h, dis, row, col, ew)


def _dot_t(a, b):
    return lax.dot_general(a, b, (((1,), (1,)), ((), ())),
                           preferred_element_type=_f32)


_R = 1000
_G = _N // _R
_DIS_R, _DIS_C = 80, 125


def _tc_proj(embx, deg3, W_w, b_w, gcn1_W):
    def body(embx_ref, deg_ref, ww_ref, bw_ref, g1w_ref, x_o, h1_o, dis_o):
        xb = _dot_t(embx_ref[...], ww_ref[...]) + bw_ref[...]
        h1 = _dot_t(xb, g1w_ref[...])
        x_o[...] = xb
        h1_o[...] = h1
        deg = deg_ref[0] + deg_ref[1] + 1.0
        dis_o[...] = lax.rsqrt(deg)

    rows_per = _DIS_R // _G
    return pl.pallas_call(
        body,
        grid=(_G,),
        in_specs=[
            pl.BlockSpec((_R, _D), lambda i: (i, 0)),
            pl.BlockSpec((2, rows_per, _DIS_C), lambda i: (0, i, 0)),
            pl.BlockSpec((_D, _D), lambda i: (0, 0)),
            pl.BlockSpec((1, _D), lambda i: (0, 0)),
            pl.BlockSpec((_D, _D), lambda i: (0, 0)),
        ],
        out_specs=[
            pl.BlockSpec((_R, _D), lambda i: (i, 0)),
            pl.BlockSpec((_R, _D), lambda i: (i, 0)),
            pl.BlockSpec((rows_per, _DIS_C), lambda i: (i, 0)),
        ],
        out_shape=[
            jax.ShapeDtypeStruct((_N, _D), _f32),
            jax.ShapeDtypeStruct((_N, _D), _f32),
            jax.ShapeDtypeStruct((_DIS_R, _DIS_C), _f32),
        ],
    )(embx, deg3, W_w, b_w, gcn1_W)


def _lstm2(x0, x1, wih, whh, bl):
    g0 = _dot_t(x0, wih) + bl
    i0 = jax.nn.sigmoid(g0[:, 0:_D])
    gg0 = jnp.tanh(g0[:, 2 * _D:3 * _D])
    o0 = jax.nn.sigmoid(g0[:, 3 * _D:4 * _D])
    c = i0 * gg0
    h = o0 * jnp.tanh(c)
    g1 = _dot_t(x1, wih) + _dot_t(h, whh) + bl
    i1 = jax.nn.sigmoid(g1[:, 0:_D])
    f1 = jax.nn.sigmoid(g1[:, _D:2 * _D])
    gg1 = jnp.tanh(g1[:, 2 * _D:3 * _D])
    o1 = jax.nn.sigmoid(g1[:, 3 * _D:4 * _D])
    c = f1 * c + i1 * gg1
    return o1 * jnp.tanh(c)


def _tc_lstm_mid(x, p, bias_conv, wih, whh, bl, wnext):
    def body(x_ref, p_ref, bc_ref, wih_ref, whh_ref, bl_ref, wn_ref,
             x1_o, h2_o):
        out1 = p_ref[0] + p_ref[1] + bc_ref[...]
        h = _lstm2(x_ref[...], out1, wih_ref[...], whh_ref[...], bl_ref[...])
        x1_o[...] = h
        h2_o[...] = _dot_t(h, wn_ref[...])

    return pl.pallas_call(
        body,
        grid=(_G,),
        in_specs=[
            pl.BlockSpec((_R, _D), lambda i: (i, 0)),
            pl.BlockSpec((2, _R, _D), lambda i: (0, i, 0)),
            pl.BlockSpec((1, _D), lambda i: (0, 0)),
            pl.BlockSpec((4 * _D, _D), lambda i: (0, 0)),
            pl.BlockSpec((4 * _D, _D), lambda i: (0, 0)),
            pl.BlockSpec((1, 4 * _D), lambda i: (0, 0)),
            pl.BlockSpec((_D, _D), lambda i: (0, 0)),
        ],
        out_specs=[
            pl.BlockSpec((_R, _D), lambda i: (i, 0)),
            pl.BlockSpec((_R, _D), lambda i: (i, 0)),
        ],
        out_shape=[
            jax.ShapeDtypeStruct((_N, _D), _f32),
            jax.ShapeDtypeStruct((_N, _D), _f32),
        ],
    )(x, p, bias_conv, wih, whh, bl, wnext)


def _tc_lstm_fin(x, p, bias_conv, wih, whh, bl, mask_col):
    def body(x_ref, p_ref, bc_ref, wih_ref, whh_ref, bl_ref, m_ref,
             x2_o, xo_o):
        out2 = p_ref[0] + p_ref[1] + bc_ref[...]
        h = _lstm2(x_ref[...], out2, wih_ref[...], whh_ref[...], bl_ref[...])
        x2_o[...] = h
        xo_o[...] = jnp.tanh(h) * m_ref[...]

    return pl.pallas_call(
        body,
        grid=(_G,),
        in_specs=[
            pl.BlockSpec((_R, _D), lambda i: (i, 0)),
            pl.BlockSpec((2, _R, _D), lambda i: (0, i, 0)),
            pl.BlockSpec((1, _D), lambda i: (0, 0)),
            pl.BlockSpec((4 * _D, _D), lambda i: (0, 0)),
            pl.BlockSpec((4 * _D, _D), lambda i: (0, 0)),
            pl.BlockSpec((1, 4 * _D), lambda i: (0, 0)),
            pl.BlockSpec((_R, 1), lambda i: (i, 0)),
        ],
        out_specs=[
            pl.BlockSpec((_R, _D), lambda i: (i, 0)),
            pl.BlockSpec((_R, _D), lambda i: (i, 0)),
        ],
        out_shape=[
            jax.ShapeDtypeStruct((_N, _D), _f32),
            jax.ShapeDtypeStruct((_N, _D), _f32),
        ],
    )(x, p, bias_conv, wih, whh, bl, mask_col)


def _tc_pool(x1, x2, ms_col, p1w, p1b, p2w, wih, whh, bl):
    def pool_one(xf, ms, p1w_v, p1b_v, p2w_v):
        hh = jnp.tanh(_dot_t(xf, p1w_v) + p1b_v)
        s = jnp.sum(hh * p2w_v, axis=1, keepdims=True) + ms
        s3 = s.reshape(_B, _L, 1)
        m = jnp.max(s3, axis=1, keepdims=True)
        e = jnp.exp(s3 - m)
        den = jnp.sum(e, axis=1, keepdims=True)
        alpha = e / den
        x3 = xf.reshape(_B, _L, _D)
        return jnp.sum(alpha * x3, axis=1)

    def body(x1_ref, x2_ref, ms_ref, p1w_ref, p1b_ref, p2w_ref,
             wih_ref, whh_ref, bl_ref, go_o):
        ms = ms_ref[...]
        g1 = pool_one(x1_ref[...], ms, p1w_ref[...], p1b_ref[...], p2w_ref[...])
        g2 = pool_one(x2_ref[...], ms, p1w_ref[...], p1b_ref[...], p2w_ref[...])
        go_o[...] = _lstm2(g1, g2, wih_ref[...], whh_ref[...], bl_ref[...])

    return pl.pallas_call(
        body,
        grid=(1,),
        in_specs=[
            pl.BlockSpec((_N, _D), lambda i: (0, 0)),
            pl.BlockSpec((_N, _D), lambda i: (0, 0)),
            pl.BlockSpec((_N, 1), lambda i: (0, 0)),
            pl.BlockSpec((_D, _D), lambda i: (0, 0)),
            pl.BlockSpec((1, _D), lambda i: (0, 0)),
            pl.BlockSpec((1, _D), lambda i: (0, 0)),
            pl.BlockSpec((4 * _D, _D), lambda i: (0, 0)),
            pl.BlockSpec((4 * _D, _D), lambda i: (0, 0)),
            pl.BlockSpec((1, 4 * _D), lambda i: (0, 0)),
        ],
        out_specs=pl.BlockSpec((_B, _D), lambda i: (0, 0)),
        out_shape=jax.ShapeDtypeStruct((_B, _D), _f32),
    )(x1, x2, ms_col, p1w, p1b, p2w, wih, whh, bl)


def kernel(x_tokens, word_type, edge_index, edge_attr, mask_pad, mask_score,
           emb_table, wt_table, W_w, b_w, gcn1_W, gcn1_b, gcn2_W, gcn2_b,
           pool1_W, pool1_b, pool2_W, lx_Wih, lx_Whh, lx_bih, lx_bhh,
           lg_Wih, lg_Whh, lg_bih, lg_bhh):
    tok = x_tokens.astype(_i32)
    wty = word_type.astype(_i32)
    ei = edge_index.astype(_i32)
    row = ei[0]
    col = ei[1]
    ew = edge_attr.astype(_f32)

    embx, deg_p = _sc_prep(tok, wty, col, ew, emb_table, wt_table)
    x, h1, dis2 = _tc_proj(embx, deg_p.reshape(2, _DIS_R, _DIS_C),
                           W_w, b_w.reshape(1, _D), gcn1_W)
    dis = dis2.reshape(_N)

    lxb = (lx_bih + lx_bhh).reshape(1, 4 * _D)
    lgb = (lg_bih + lg_bhh).reshape(1, 4 * _D)

    p = _sc_conv(h1, dis, row, col, ew).reshape(2, _N, _D)
    x1, h2 = _tc_lstm_mid(x, p, gcn1_b.reshape(1, _D),
                          lx_Wih, lx_Whh, lxb, gcn2_W)

    q = _sc_conv(h2, dis, row, col, ew).reshape(2, _N, _D)
    x2, xo = _tc_lstm_fin(x1, q, gcn2_b.reshape(1, _D),
                          lx_Wih, lx_Whh, lxb, mask_pad.reshape(_N, 1))

    g_out = _tc_pool(x1, x2, mask_score.reshape(_N, 1),
                     pool1_W, pool1_b.reshape(1, _D), pool2_W,
                     lg_Wih, lg_Whh, lgb)

    return (xo.reshape(_B, _L, _D), g_out)


# speedup vs baseline: 12.9360x; 1.3318x over previous
import functools

import jax
import jax.numpy as jnp
from jax import lax
from jax.experimental import pallas as pl
from jax.experimental.pallas import tpu as pltpu
from jax.experimental.pallas import tpu_sc as plsc

_N = 10000
_E = 320000
_B = 50
_L = 200
_D = 128

_NC = 2
_NS = 16
_NW = _NC * _NS

_RC = 80
_NCH = _N // _RC
_PE = _E // _NW
_DC = 2000
_EC = 80

_f32 = jnp.float32
_i32 = jnp.int32


def _scale_rows(buf, sbuf, n):
    def body(g, _):
        sv = sbuf[pl.ds(g * 16, 16)]
        for l in range(16):
            i = g * 16 + l
            s = sv[l]
            for j in range(_D // 16):
                sl = pl.ds(j * 16, 16)
                buf[i, sl] = buf[i, sl] * s
        return 0
    lax.fori_loop(0, n // 16, body, 0)


def _zero_rows(buf, n):
    z = jnp.zeros((16,), _f32)
    def body(i, _):
        for j in range(_D // 16):
            buf[i, pl.ds(j * 16, 16)] = z
        return 0
    lax.fori_loop(0, n, body, 0)


def _sc_prep(tok, wty, col, ew, emb_table, wt_table):
    mesh = plsc.VectorSubcoreMesh(core_axis_name="c", subcore_axis_name="s")

    @functools.partial(
        pl.kernel,
        mesh=mesh,
        out_type=(
            jax.ShapeDtypeStruct((_N, _D), _f32),
            jax.ShapeDtypeStruct((2 * _N,), _f32),
        ),
        scratch_types=[
            pltpu.VMEM((_RC,), _i32),
            pltpu.VMEM((_RC,), _i32),
            pltpu.VMEM((_RC, _D), _f32),
            pltpu.VMEM((_RC, _D), _f32),
            pltpu.VMEM((_DC,), _i32),
            pltpu.VMEM((_DC,), _f32),
            pltpu.VMEM_SHARED((_N,), _f32),
            pltpu.SemaphoreType.DMA,
            pltpu.SemaphoreType.DMA,
        ],
    )
    def k(tok_h, wty_h, col_h, ew_h, emb_h, wt_h, embx_o, deg_o,
          tokbuf, wtbuf, ebuf, wbuf, colbuf, ewbuf, deg_acc, sem1, sem2):
        cid = lax.axis_index("c")
        sid = lax.axis_index("s")
        w = sid * _NC + cid

        def zb(i, _):
            ewbuf[pl.ds(i * 16, 16)] = jnp.zeros((16,), _f32)
            return 0
        lax.fori_loop(0, _DC // 16, zb, 0)

        @pl.when(sid < _N // _DC)
        def _():
            pltpu.sync_copy(ewbuf, deg_acc.at[pl.ds(sid * _DC, _DC)])
        plsc.subcore_barrier()

        def dchunk(kk, _):
            base = w * _PE + kk * _DC
            pltpu.sync_copy(col_h.at[pl.ds(base, _DC)], colbuf)
            pltpu.sync_copy(ew_h.at[pl.ds(base, _DC)], ewbuf)
            pltpu.sync_copy(ewbuf, deg_acc.at[colbuf], add=True)
            return 0
        lax.fori_loop(0, _PE // _DC, dchunk, 0)
        plsc.subcore_barrier()

        @pl.when(sid < _N // _DC)
        def _():
            pltpu.sync_copy(deg_acc.at[pl.ds(sid * _DC, _DC)], ewbuf)
            pltpu.sync_copy(ewbuf, deg_o.at[pl.ds(cid * _N + sid * _DC, _DC)])

        for r in range((_NCH + _NW - 1) // _NW):
            ck = w + _NW * r

            @pl.when(ck < _NCH)
            def _():
                base = ck * _RC
                pltpu.sync_copy(tok_h.at[pl.ds(base, _RC)], tokbuf)
                pltpu.sync_copy(wty_h.at[pl.ds(base, _RC)], wtbuf)
                cp1 = pltpu.async_copy(emb_h.at[tokbuf], ebuf, sem1)
                cp2 = pltpu.async_copy(wt_h.at[wtbuf], wbuf, sem2)
                cp1.wait()
                cp2.wait()

                def addb(i, _):
                    for j in range(_D // 16):
                        sl = pl.ds(j * 16, 16)
                        ebuf[i, sl] = ebuf[i, sl] + wbuf[i, sl]
                    return 0
                lax.fori_loop(0, _RC, addb, 0)
                pltpu.sync_copy(ebuf, embx_o.at[pl.ds(base, _RC)])

    return k(tok, wty, col, ew, emb_table, wt_table)


def _sc_conv(h, dis, row, col, ew):
    mesh = plsc.VectorSubcoreMesh(core_axis_name="c", subcore_axis_name="s")

    @functools.partial(
        pl.kernel,
        mesh=mesh,
        out_type=jax.ShapeDtypeStruct((2 * _N, _D), _f32),
        scratch_types=[
            pltpu.VMEM((_EC,), _i32),
            pltpu.VMEM((_EC,), _i32),
            pltpu.VMEM((_EC,), _i32),
            pltpu.VMEM((_EC,), _i32),
            pltpu.VMEM((_EC,), _f32),
            pltpu.VMEM((_EC,), _f32),
            pltpu.VMEM((_EC,), _f32),
            pltpu.VMEM((_EC,), _f32),
            pltpu.VMEM((_EC, _D), _f32),
            pltpu.VMEM((_EC, _D), _f32),
            pltpu.VMEM_SHARED((_N, _D), _f32),
            pltpu.SemaphoreType.DMA,
            pltpu.SemaphoreType.DMA,
            pltpu.SemaphoreType.DMA,
            pltpu.SemaphoreType.DMA,
        ],
    )
    def k(h_h, dis_h, row_h, col_h, ew_h, out_o,
          rowb0, rowb1, colb0, colb1, ewb0, ewb1, dgb0, dgb1, rsb0, rsb1,
          acc, gs0, gs1, ds0, ds1):
        cid = lax.axis_index("c")
        sid = lax.axis_index("s")
        w = sid * _NC + cid
        rowb = (rowb0, rowb1)
        colb = (colb0, colb1)
        ewb = (ewb0, ewb1)
        dgb = (dgb0, dgb1)
        rsb = (rsb0, rsb1)
        gs = (gs0, gs1)
        ds = (ds0, ds1)

        @pl.when(cid != 0)
        def _():
            _zero_rows(rsb0, _RC)

        for r in range((_NCH + _NS - 1) // _NS):
            ck = sid + _NS * r

            @pl.when(ck < _NCH)
            def _():
                base = ck * _RC

                @pl.when(cid == 0)
                def _():
                    pltpu.sync_copy(h_h.at[pl.ds(base, _RC)], rsb0)
                    pltpu.sync_copy(dis_h.at[pl.ds(base, _RC)], dgb0)
                    _scale_rows(rsb0, dgb0, _RC)
                pltpu.sync_copy(rsb0, acc.at[pl.ds(base, _RC)])
        plsc.subcore_barrier()

        nch = _PE // _EC

        def issue(q, kk):
            base = w * _PE + kk * _EC
            pltpu.sync_copy(row_h.at[pl.ds(base, _EC)], rowb[q])
            pltpu.sync_copy(ew_h.at[pl.ds(base, _EC)], ewb[q])
            pltpu.sync_copy(col_h.at[pl.ds(base, _EC)], colb[q])
            pltpu.async_copy(h_h.at[rowb[q]], rsb[q], gs[q])
            pltpu.async_copy(dis_h.at[rowb[q]], dgb[q], ds[q])

        def process(q):
            pltpu.make_async_copy(h_h.at[rowb[q]], rsb[q], gs[q]).wait()
            pltpu.make_async_copy(dis_h.at[rowb[q]], dgb[q], ds[q]).wait()

            def sb(g, _):
                sv = ewb[q][pl.ds(g * 16, 16)] * dgb[q][pl.ds(g * 16, 16)]
                for l in range(16):
                    i = g * 16 + l
                    s = sv[l]
                    for j in range(_D // 16):
                        sl = pl.ds(j * 16, 16)
                        rsb[q][i, sl] = rsb[q][i, sl] * s
                return 0
            lax.fori_loop(0, _EC // 16, sb, 0)
            pltpu.sync_copy(rsb[q], acc.at[colb[q]], add=True)

        issue(0, 0)

        def epair(ko, _):
            issue(1, 2 * ko + 1)
            process(0)
            issue(0, 2 * ko + 2)
            process(1)
            return 0
        lax.fori_loop(0, (nch - 1) // 2, epair, 0)
        process(0)
        plsc.subcore_barrier()

        for r in range((_NCH + _NS - 1) // _NS):
            ck = sid + _NS * r

            @pl.when(ck < _NCH)
            def _():
                base = ck * _RC
                pltpu.sync_copy(acc.at[pl.ds(base, _RC)], rsb0)
                pltpu.sync_copy(dis_h.at[pl.ds(base, _RC)], dgb0)
                _scale_rows(rsb0, dgb0, _RC)
                pltpu.sync_copy(rsb0, out_o.at[pl.ds(cid * _N + base, _RC)])

    return k(h, dis, row, col, ew)


def _dot_t(a, b):
    return lax.dot_general(a, b, (((1,), (1,)), ((), ())),
                           preferred_element_type=_f32)


_R = 1000
_G = _N // _R
_DIS_R, _DIS_C = 80, 125


def _tc_proj(embx, deg3, W_w, b_w, gcn1_W):
    def body(embx_ref, deg_ref, ww_ref, bw_ref, g1w_ref, x_o, h1_o, dis_o):
        xb = _dot_t(embx_ref[...], ww_ref[...]) + bw_ref[...]
        h1 = _dot_t(xb, g1w_ref[...])
        x_o[...] = xb
        h1_o[...] = h1
        deg = deg_ref[0] + deg_ref[1] + 1.0
        dis_o[...] = lax.rsqrt(deg)

    rows_per = _DIS_R // _G
    return pl.pallas_call(
        body,
        grid=(_G,),
        in_specs=[
            pl.BlockSpec((_R, _D), lambda i: (i, 0)),
            pl.BlockSpec((2, rows_per, _DIS_C), lambda i: (0, i, 0)),
            pl.BlockSpec((_D, _D), lambda i: (0, 0)),
            pl.BlockSpec((1, _D), lambda i: (0, 0)),
            pl.BlockSpec((_D, _D), lambda i: (0, 0)),
        ],
        out_specs=[
            pl.BlockSpec((_R, _D), lambda i: (i, 0)),
            pl.BlockSpec((_R, _D), lambda i: (i, 0)),
            pl.BlockSpec((rows_per, _DIS_C), lambda i: (i, 0)),
        ],
        out_shape=[
            jax.ShapeDtypeStruct((_N, _D), _f32),
            jax.ShapeDtypeStruct((_N, _D), _f32),
            jax.ShapeDtypeStruct((_DIS_R, _DIS_C), _f32),
        ],
    )(embx, deg3, W_w, b_w, gcn1_W)


def _lstm2(x0, x1, wih, whh, bl):
    g0 = _dot_t(x0, wih) + bl
    i0 = jax.nn.sigmoid(g0[:, 0:_D])
    gg0 = jnp.tanh(g0[:, 2 * _D:3 * _D])
    o0 = jax.nn.sigmoid(g0[:, 3 * _D:4 * _D])
    c = i0 * gg0
    h = o0 * jnp.tanh(c)
    g1 = _dot_t(x1, wih) + _dot_t(h, whh) + bl
    i1 = jax.nn.sigmoid(g1[:, 0:_D])
    f1 = jax.nn.sigmoid(g1[:, _D:2 * _D])
    gg1 = jnp.tanh(g1[:, 2 * _D:3 * _D])
    o1 = jax.nn.sigmoid(g1[:, 3 * _D:4 * _D])
    c = f1 * c + i1 * gg1
    return o1 * jnp.tanh(c)


def _tc_lstm_mid(x, p, bias_conv, wih, whh, bl, wnext):
    def body(x_ref, p_ref, bc_ref, wih_ref, whh_ref, bl_ref, wn_ref,
             x1_o, h2_o):
        out1 = p_ref[0] + p_ref[1] + bc_ref[...]
        h = _lstm2(x_ref[...], out1, wih_ref[...], whh_ref[...], bl_ref[...])
        x1_o[...] = h
        h2_o[...] = _dot_t(h, wn_ref[...])

    return pl.pallas_call(
        body,
        grid=(_G,),
        in_specs=[
            pl.BlockSpec((_R, _D), lambda i: (i, 0)),
            pl.BlockSpec((2, _R, _D), lambda i: (0, i, 0)),
            pl.BlockSpec((1, _D), lambda i: (0, 0)),
            pl.BlockSpec((4 * _D, _D), lambda i: (0, 0)),
            pl.BlockSpec((4 * _D, _D), lambda i: (0, 0)),
            pl.BlockSpec((1, 4 * _D), lambda i: (0, 0)),
            pl.BlockSpec((_D, _D), lambda i: (0, 0)),
        ],
        out_specs=[
            pl.BlockSpec((_R, _D), lambda i: (i, 0)),
            pl.BlockSpec((_R, _D), lambda i: (i, 0)),
        ],
        out_shape=[
            jax.ShapeDtypeStruct((_N, _D), _f32),
            jax.ShapeDtypeStruct((_N, _D), _f32),
        ],
    )(x, p, bias_conv, wih, whh, bl, wnext)


def _tc_lstm_fin(x, p, bias_conv, wih, whh, bl, mask_col):
    def body(x_ref, p_ref, bc_ref, wih_ref, whh_ref, bl_ref, m_ref,
             x2_o, xo_o):
        out2 = p_ref[0] + p_ref[1] + bc_ref[...]
        h = _lstm2(x_ref[...], out2, wih_ref[...], whh_ref[...], bl_ref[...])
        x2_o[...] = h
        xo_o[...] = jnp.tanh(h) * m_ref[...]

    return pl.pallas_call(
        body,
        grid=(_G,),
        in_specs=[
            pl.BlockSpec((_R, _D), lambda i: (i, 0)),
            pl.BlockSpec((2, _R, _D), lambda i: (0, i, 0)),
            pl.BlockSpec((1, _D), lambda i: (0, 0)),
            pl.BlockSpec((4 * _D, _D), lambda i: (0, 0)),
            pl.BlockSpec((4 * _D, _D), lambda i: (0, 0)),
            pl.BlockSpec((1, 4 * _D), lambda i: (0, 0)),
            pl.BlockSpec((_R, 1), lambda i: (i, 0)),
        ],
        out_specs=[
            pl.BlockSpec((_R, _D), lambda i: (i, 0)),
            pl.BlockSpec((_R, _D), lambda i: (i, 0)),
        ],
        out_shape=[
            jax.ShapeDtypeStruct((_N, _D), _f32),
            jax.ShapeDtypeStruct((_N, _D), _f32),
        ],
    )(x, p, bias_conv, wih, whh, bl, mask_col)


def _tc_pool(x1, x2, ms_col, p1w, p1b, p2w, wih, whh, bl):
    def pool_one(xf, ms, p1w_v, p1b_v, p2w_v):
        hh = jnp.tanh(_dot_t(xf, p1w_v) + p1b_v)
        s = jnp.sum(hh * p2w_v, axis=1, keepdims=True) + ms
        s3 = s.reshape(_B, _L, 1)
        m = jnp.max(s3, axis=1, keepdims=True)
        e = jnp.exp(s3 - m)
        den = jnp.sum(e, axis=1, keepdims=True)
        alpha = e / den
        x3 = xf.reshape(_B, _L, _D)
        return jnp.sum(alpha * x3, axis=1)

    def body(x1_ref, x2_ref, ms_ref, p1w_ref, p1b_ref, p2w_ref,
             wih_ref, whh_ref, bl_ref, go_o):
        ms = ms_ref[...]
        g1 = pool_one(x1_ref[...], ms, p1w_ref[...], p1b_ref[...], p2w_ref[...])
        g2 = pool_one(x2_ref[...], ms, p1w_ref[...], p1b_ref[...], p2w_ref[...])
        go_o[...] = _lstm2(g1, g2, wih_ref[...], whh_ref[...], bl_ref[...])

    return pl.pallas_call(
        body,
        grid=(1,),
        in_specs=[
            pl.BlockSpec((_N, _D), lambda i: (0, 0)),
            pl.BlockSpec((_N, _D), lambda i: (0, 0)),
            pl.BlockSpec((_N, 1), lambda i: (0, 0)),
            pl.BlockSpec((_D, _D), lambda i: (0, 0)),
            pl.BlockSpec((1, _D), lambda i: (0, 0)),
            pl.BlockSpec((1, _D), lambda i: (0, 0)),
            pl.BlockSpec((4 * _D, _D), lambda i: (0, 0)),
            pl.BlockSpec((4 * _D, _D), lambda i: (0, 0)),
            pl.BlockSpec((1, 4 * _D), lambda i: (0, 0)),
        ],
        out_specs=pl.BlockSpec((_B, _D), lambda i: (0, 0)),
        out_shape=jax.ShapeDtypeStruct((_B, _D), _f32),
    )(x1, x2, ms_col, p1w, p1b, p2w, wih, whh, bl)


def kernel(x_tokens, word_type, edge_index, edge_attr, mask_pad, mask_score,
           emb_table, wt_table, W_w, b_w, gcn1_W, gcn1_b, gcn2_W, gcn2_b,
           pool1_W, pool1_b, pool2_W, lx_Wih, lx_Whh, lx_bih, lx_bhh,
           lg_Wih, lg_Whh, lg_bih, lg_bhh):
    tok = x_tokens.astype(_i32)
    wty = word_type.astype(_i32)
    ei = edge_index.astype(_i32)
    row = ei[0]
    col = ei[1]
    ew = edge_attr.astype(_f32)

    embx, deg_p = _sc_prep(tok, wty, col, ew, emb_table, wt_table)
    x, h1, dis2 = _tc_proj(embx, deg_p.reshape(2, _DIS_R, _DIS_C),
                           W_w, b_w.reshape(1, _D), gcn1_W)
    dis = dis2.reshape(_N)

    lxb = (lx_bih + lx_bhh).reshape(1, 4 * _D)
    lgb = (lg_bih + lg_bhh).reshape(1, 4 * _D)

    p = _sc_conv(h1, dis, row, col, ew).reshape(2, _N, _D)
    x1, h2 = _tc_lstm_mid(x, p, gcn1_b.reshape(1, _D),
                          lx_Wih, lx_Whh, lxb, gcn2_W)

    q = _sc_conv(h2, dis, row, col, ew).reshape(2, _N, _D)
    x2, xo = _tc_lstm_fin(x1, q, gcn2_b.reshape(1, _D),
                          lx_Wih, lx_Whh, lxb, mask_pad.reshape(_N, 1))

    g_out = _tc_pool(x1, x2, mask_score.reshape(_N, 1),
                     pool1_W, pool1_b.reshape(1, _D), pool2_W,
                     lg_Wih, lg_Whh, lgb)

    return (xo.reshape(_B, _L, _D), g_out)


# speedup vs baseline: 19.3709x; 1.4974x over previous
import functools

import jax
import jax.numpy as jnp
from jax import lax
from jax.experimental import pallas as pl
from jax.experimental.pallas import tpu as pltpu
from jax.experimental.pallas import tpu_sc as plsc

_N = 10000
_E = 320000
_B = 50
_L = 200
_D = 128

_NC = 2
_NS = 16
_NW = _NC * _NS

_RC = 80
_NCH = _N // _RC
_PE = _E // _NW
_DC = 2000
_EC = 80

_f32 = jnp.float32
_i32 = jnp.int32


def _scale_rows(buf, sbuf, n):
    def body(g, _):
        sv = sbuf[pl.ds(g * 16, 16)]
        for l in range(16):
            i = g * 16 + l
            s = sv[l]
            for j in range(_D // 16):
                sl = pl.ds(j * 16, 16)
                buf[i, sl] = buf[i, sl] * s
        return 0
    lax.fori_loop(0, n // 16, body, 0)


def _zero_rows(buf, n):
    z = jnp.zeros((16,), _f32)
    def body(i, _):
        for j in range(_D // 16):
            buf[i, pl.ds(j * 16, 16)] = z
        return 0
    lax.fori_loop(0, n, body, 0)


def _sc_prep(tok, wty, col, ew, emb_table, wt_table):
    mesh = plsc.VectorSubcoreMesh(core_axis_name="c", subcore_axis_name="s")

    @functools.partial(
        pl.kernel,
        mesh=mesh,
        out_type=(
            jax.ShapeDtypeStruct((_N, _D), _f32),
            jax.ShapeDtypeStruct((2 * _N,), _f32),
        ),
        scratch_types=[
            pltpu.VMEM((_RC,), _i32),
            pltpu.VMEM((_RC,), _i32),
            pltpu.VMEM((_RC, _D), _f32),
            pltpu.VMEM((_RC, _D), _f32),
            pltpu.VMEM((_DC,), _i32),
            pltpu.VMEM((_DC,), _f32),
            pltpu.VMEM_SHARED((_N,), _f32),
            pltpu.SemaphoreType.DMA,
            pltpu.SemaphoreType.DMA,
        ],
    )
    def k(tok_h, wty_h, col_h, ew_h, emb_h, wt_h, embx_o, deg_o,
          tokbuf, wtbuf, ebuf, wbuf, colbuf, ewbuf, deg_acc, sem1, sem2):
        cid = lax.axis_index("c")
        sid = lax.axis_index("s")
        w = sid * _NC + cid

        def zb(i, _):
            ewbuf[pl.ds(i * 16, 16)] = jnp.zeros((16,), _f32)
            return 0
        lax.fori_loop(0, _DC // 16, zb, 0)

        @pl.when(sid < _N // _DC)
        def _():
            pltpu.sync_copy(ewbuf, deg_acc.at[pl.ds(sid * _DC, _DC)])
        plsc.subcore_barrier()

        def dchunk(kk, _):
            base = w * _PE + kk * _DC
            pltpu.sync_copy(col_h.at[pl.ds(base, _DC)], colbuf)
            pltpu.sync_copy(ew_h.at[pl.ds(base, _DC)], ewbuf)
            pltpu.sync_copy(ewbuf, deg_acc.at[colbuf], add=True)
            return 0
        lax.fori_loop(0, _PE // _DC, dchunk, 0)
        plsc.subcore_barrier()

        @pl.when(sid < _N // _DC)
        def _():
            pltpu.sync_copy(deg_acc.at[pl.ds(sid * _DC, _DC)], ewbuf)
            pltpu.sync_copy(ewbuf, deg_o.at[pl.ds(cid * _N + sid * _DC, _DC)])

        for r in range((_NCH + _NW - 1) // _NW):
            ck = w + _NW * r

            @pl.when(ck < _NCH)
            def _():
                base = ck * _RC
                pltpu.sync_copy(tok_h.at[pl.ds(base, _RC)], tokbuf)
                pltpu.sync_copy(wty_h.at[pl.ds(base, _RC)], wtbuf)
                cp1 = pltpu.async_copy(emb_h.at[tokbuf], ebuf, sem1)
                cp2 = pltpu.async_copy(wt_h.at[wtbuf], wbuf, sem2)
                cp1.wait()
                cp2.wait()

                def addb(i, _):
                    for j in range(_D // 16):
                        sl = pl.ds(j * 16, 16)
                        ebuf[i, sl] = ebuf[i, sl] + wbuf[i, sl]
                    return 0
                lax.fori_loop(0, _RC, addb, 0)
                pltpu.sync_copy(ebuf, embx_o.at[pl.ds(base, _RC)])

    return k(tok, wty, col, ew, emb_table, wt_table)


def _sc_conv(h, dis, row, col, ew):
    mesh = plsc.VectorSubcoreMesh(core_axis_name="c", subcore_axis_name="s")

    @functools.partial(
        pl.kernel,
        mesh=mesh,
        out_type=jax.ShapeDtypeStruct((2 * _N, _D), _f32),
        scratch_types=[
            pltpu.VMEM((_PE,), _i32),
            pltpu.VMEM((_PE,), _f32),
            pltpu.VMEM((_EC,), _i32),
            pltpu.VMEM((_EC,), _i32),
            pltpu.VMEM((_EC,), _f32),
            pltpu.VMEM((_EC,), _f32),
            pltpu.VMEM((_EC, _D), _f32),
            pltpu.VMEM((_EC, _D), _f32),
            pltpu.VMEM_SHARED((_N, _D), _f32),
            pltpu.SemaphoreType.DMA,
            pltpu.SemaphoreType.DMA,
            pltpu.SemaphoreType.DMA,
            pltpu.SemaphoreType.DMA,
            pltpu.SemaphoreType.DMA,
            pltpu.SemaphoreType.DMA,
            pltpu.SemaphoreType.DMA,
            pltpu.SemaphoreType.DMA,
        ],
    )
    def k(h_h, dis_h, row_h, col_h, ew_h, out_o,
          rowbig, ewbig, colb0, colb1, dgb0, dgb1, rsb0, rsb1,
          acc, gs0, gs1, ds0, ds1, cs0, cs1, ss0, ss1):
        cid = lax.axis_index("c")
        sid = lax.axis_index("s")
        w = sid * _NC + cid
        colb = (colb0, colb1)
        dgb = (dgb0, dgb1)
        rsb = (rsb0, rsb1)
        gs = (gs0, gs1)
        ds = (ds0, ds1)
        cs = (cs0, cs1)
        ss = (ss0, ss1)

        @pl.when(cid != 0)
        def _():
            _zero_rows(rsb0, _RC)

        for r in range((_NCH + _NS - 1) // _NS):
            ck = sid + _NS * r

            @pl.when(ck < _NCH)
            def _():
                base = ck * _RC

                @pl.when(cid == 0)
                def _():
                    pltpu.sync_copy(h_h.at[pl.ds(base, _RC)], rsb0)
                    pltpu.sync_copy(dis_h.at[pl.ds(base, _RC)], dgb0)
                    _scale_rows(rsb0, dgb0, _RC)
                pltpu.sync_copy(rsb0, acc.at[pl.ds(base, _RC)])
        plsc.subcore_barrier()

        nch = _PE // _EC

        pltpu.sync_copy(row_h.at[pl.ds(w * _PE, _PE)], rowbig)
        pltpu.sync_copy(ew_h.at[pl.ds(w * _PE, _PE)], ewbig)

        def issue(q, kk):
            idx = rowbig.at[pl.ds(kk * _EC, _EC)]
            pltpu.async_copy(h_h.at[idx], rsb[q], gs[q])
            pltpu.async_copy(dis_h.at[idx], dgb[q], ds[q])
            pltpu.async_copy(col_h.at[pl.ds(w * _PE + kk * _EC, _EC)],
                             colb[q], cs[q])

        def wait_gathers(q, kk):
            idx = rowbig.at[pl.ds(kk * _EC, _EC)]
            pltpu.make_async_copy(h_h.at[idx], rsb[q], gs[q]).wait()
            pltpu.make_async_copy(dis_h.at[idx], dgb[q], ds[q]).wait()

        def scale(q, kk):
            def sb(g, _):
                sv = ewbig[pl.ds(kk * _EC + g * 16, 16)] * dgb[q][pl.ds(g * 16, 16)]
                for l in range(16):
                    i = g * 16 + l
                    s = sv[l]
                    for j in range(_D // 16):
                        sl = pl.ds(j * 16, 16)
                        rsb[q][i, sl] = rsb[q][i, sl] * s
                return 0
            lax.fori_loop(0, _EC // 16, sb, 0)

        def wait_col(q, kk):
            pltpu.make_async_copy(
                col_h.at[pl.ds(w * _PE + kk * _EC, _EC)], colb[q], cs[q]).wait()

        def scatter_async(q):
            pltpu.async_copy(rsb[q], acc.at[colb[q]], ss[q], add=True)

        def wait_scatter(q):
            pltpu.make_async_copy(rsb[q], acc.at[colb[q]], ss[q]).wait()

        issue(0, 0)
        issue(1, 1)

        def epair(ko, _):
            k0 = 2 * ko
            wait_gathers(0, k0)
            scale(0, k0)
            wait_col(0, k0)
            scatter_async(0)
            wait_gathers(1, k0 + 1)
            scale(1, k0 + 1)
            wait_scatter(0)
            issue(0, k0 + 2)
            wait_col(1, k0 + 1)
            scatter_async(1)
            wait_scatter(1)
            issue(1, k0 + 3)
            return 0
        lax.fori_loop(0, (nch - 3) // 2, epair, 0)

        kl = nch - 3
        wait_gathers(0, kl)
        scale(0, kl)
        wait_col(0, kl)
        scatter_async(0)
        wait_scatter(0)
        issue(0, kl + 2)
        wait_gathers(1, kl + 1)
        scale(1, kl + 1)
        wait_col(1, kl + 1)
        scatter_async(1)
        wait_scatter(1)
        wait_gathers(0, kl + 2)
        scale(0, kl + 2)
        wait_col(0, kl + 2)
        scatter_async(0)
        wait_scatter(0)
        plsc.subcore_barrier()

        for r in range((_NCH + _NS - 1) // _NS):
            ck = sid + _NS * r

            @pl.when(ck < _NCH)
            def _():
                base = ck * _RC
                pltpu.sync_copy(acc.at[pl.ds(base, _RC)], rsb0)
                pltpu.sync_copy(dis_h.at[pl.ds(base, _RC)], dgb0)
                _scale_rows(rsb0, dgb0, _RC)
                pltpu.sync_copy(rsb0, out_o.at[pl.ds(cid * _N + base, _RC)])

    return k(h, dis, row, col, ew)


def _dot_t(a, b):
    return lax.dot_general(a, b, (((1,), (1,)), ((), ())),
                           preferred_element_type=_f32)


_R = 1000
_G = _N // _R
_DIS_R, _DIS_C = 80, 125


def _tc_proj(embx, deg3, W_w, b_w, gcn1_W):
    def body(embx_ref, deg_ref, ww_ref, bw_ref, g1w_ref, x_o, h1_o, dis_o):
        xb = _dot_t(embx_ref[...], ww_ref[...]) + bw_ref[...]
        h1 = _dot_t(xb, g1w_ref[...])
        x_o[...] = xb
        h1_o[...] = h1
        deg = deg_ref[0] + deg_ref[1] + 1.0
        dis_o[...] = lax.rsqrt(deg)

    rows_per = _DIS_R // _G
    return pl.pallas_call(
        body,
        grid=(_G,),
        in_specs=[
            pl.BlockSpec((_R, _D), lambda i: (i, 0)),
            pl.BlockSpec((2, rows_per, _DIS_C), lambda i: (0, i, 0)),
            pl.BlockSpec((_D, _D), lambda i: (0, 0)),
            pl.BlockSpec((1, _D), lambda i: (0, 0)),
            pl.BlockSpec((_D, _D), lambda i: (0, 0)),
        ],
        out_specs=[
            pl.BlockSpec((_R, _D), lambda i: (i, 0)),
            pl.BlockSpec((_R, _D), lambda i: (i, 0)),
            pl.BlockSpec((rows_per, _DIS_C), lambda i: (i, 0)),
        ],
        out_shape=[
            jax.ShapeDtypeStruct((_N, _D), _f32),
            jax.ShapeDtypeStruct((_N, _D), _f32),
            jax.ShapeDtypeStruct((_DIS_R, _DIS_C), _f32),
        ],
    )(embx, deg3, W_w, b_w, gcn1_W)


def _lstm2(x0, x1, wih, whh, bl):
    g0 = _dot_t(x0, wih) + bl
    i0 = jax.nn.sigmoid(g0[:, 0:_D])
    gg0 = jnp.tanh(g0[:, 2 * _D:3 * _D])
    o0 = jax.nn.sigmoid(g0[:, 3 * _D:4 * _D])
    c = i0 * gg0
    h = o0 * jnp.tanh(c)
    g1 = _dot_t(x1, wih) + _dot_t(h, whh) + bl
    i1 = jax.nn.sigmoid(g1[:, 0:_D])
    f1 = jax.nn.sigmoid(g1[:, _D:2 * _D])
    gg1 = jnp.tanh(g1[:, 2 * _D:3 * _D])
    o1 = jax.nn.sigmoid(g1[:, 3 * _D:4 * _D])
    c = f1 * c + i1 * gg1
    return o1 * jnp.tanh(c)


def _tc_lstm_mid(x, p, bias_conv, wih, whh, bl, wnext):
    def body(x_ref, p_ref, bc_ref, wih_ref, whh_ref, bl_ref, wn_ref,
             x1_o, h2_o):
        out1 = p_ref[0] + p_ref[1] + bc_ref[...]
        h = _lstm2(x_ref[...], out1, wih_ref[...], whh_ref[...], bl_ref[...])
        x1_o[...] = h
        h2_o[...] = _dot_t(h, wn_ref[...])

    return pl.pallas_call(
        body,
        grid=(_G,),
        in_specs=[
            pl.BlockSpec((_R, _D), lambda i: (i, 0)),
            pl.BlockSpec((2, _R, _D), lambda i: (0, i, 0)),
            pl.BlockSpec((1, _D), lambda i: (0, 0)),
            pl.BlockSpec((4 * _D, _D), lambda i: (0, 0)),
            pl.BlockSpec((4 * _D, _D), lambda i: (0, 0)),
            pl.BlockSpec((1, 4 * _D), lambda i: (0, 0)),
            pl.BlockSpec((_D, _D), lambda i: (0, 0)),
        ],
        out_specs=[
            pl.BlockSpec((_R, _D), lambda i: (i, 0)),
            pl.BlockSpec((_R, _D), lambda i: (i, 0)),
        ],
        out_shape=[
            jax.ShapeDtypeStruct((_N, _D), _f32),
            jax.ShapeDtypeStruct((_N, _D), _f32),
        ],
    )(x, p, bias_conv, wih, whh, bl, wnext)


def _tc_lstm_fin(x, p, bias_conv, wih, whh, bl, mask_col):
    def body(x_ref, p_ref, bc_ref, wih_ref, whh_ref, bl_ref, m_ref,
             x2_o, xo_o):
        out2 = p_ref[0] + p_ref[1] + bc_ref[...]
        h = _lstm2(x_ref[...], out2, wih_ref[...], whh_ref[...], bl_ref[...])
        x2_o[...] = h
        xo_o[...] = jnp.tanh(h) * m_ref[...]

    return pl.pallas_call(
        body,
        grid=(_G,),
        in_specs=[
            pl.BlockSpec((_R, _D), lambda i: (i, 0)),
            pl.BlockSpec((2, _R, _D), lambda i: (0, i, 0)),
            pl.BlockSpec((1, _D), lambda i: (0, 0)),
            pl.BlockSpec((4 * _D, _D), lambda i: (0, 0)),
            pl.BlockSpec((4 * _D, _D), lambda i: (0, 0)),
            pl.BlockSpec((1, 4 * _D), lambda i: (0, 0)),
            pl.BlockSpec((_R, 1), lambda i: (i, 0)),
        ],
        out_specs=[
            pl.BlockSpec((_R, _D), lambda i: (i, 0)),
            pl.BlockSpec((_R, _D), lambda i: (i, 0)),
        ],
        out_shape=[
            jax.ShapeDtypeStruct((_N, _D), _f32),
            jax.ShapeDtypeStruct((_N, _D), _f32),
        ],
    )(x, p, bias_conv, wih, whh, bl, mask_col)


def _tc_pool(x1, x2, ms_col, p1w, p1b, p2w, wih, whh, bl):
    def pool_one(xf, ms, p1w_v, p1b_v, p2w_v):
        hh = jnp.tanh(_dot_t(xf, p1w_v) + p1b_v)
        s = jnp.sum(hh * p2w_v, axis=1, keepdims=True) + ms
        s3 = s.reshape(_B, _L, 1)
        m = jnp.max(s3, axis=1, keepdims=True)
        e = jnp.exp(s3 - m)
        den = jnp.sum(e, axis=1, keepdims=True)
        alpha = e / den
        x3 = xf.reshape(_B, _L, _D)
        return jnp.sum(alpha * x3, axis=1)

    def body(x1_ref, x2_ref, ms_ref, p1w_ref, p1b_ref, p2w_ref,
             wih_ref, whh_ref, bl_ref, go_o):
        ms = ms_ref[...]
        g1 = pool_one(x1_ref[...], ms, p1w_ref[...], p1b_ref[...], p2w_ref[...])
        g2 = pool_one(x2_ref[...], ms, p1w_ref[...], p1b_ref[...], p2w_ref[...])
        go_o[...] = _lstm2(g1, g2, wih_ref[...], whh_ref[...], bl_ref[...])

    return pl.pallas_call(
        body,
        grid=(1,),
        in_specs=[
            pl.BlockSpec((_N, _D), lambda i: (0, 0)),
            pl.BlockSpec((_N, _D), lambda i: (0, 0)),
            pl.BlockSpec((_N, 1), lambda i: (0, 0)),
            pl.BlockSpec((_D, _D), lambda i: (0, 0)),
            pl.BlockSpec((1, _D), lambda i: (0, 0)),
            pl.BlockSpec((1, _D), lambda i: (0, 0)),
            pl.BlockSpec((4 * _D, _D), lambda i: (0, 0)),
            pl.BlockSpec((4 * _D, _D), lambda i: (0, 0)),
            pl.BlockSpec((1, 4 * _D), lambda i: (0, 0)),
        ],
        out_specs=pl.BlockSpec((_B, _D), lambda i: (0, 0)),
        out_shape=jax.ShapeDtypeStruct((_B, _D), _f32),
    )(x1, x2, ms_col, p1w, p1b, p2w, wih, whh, bl)


def kernel(x_tokens, word_type, edge_index, edge_attr, mask_pad, mask_score,
           emb_table, wt_table, W_w, b_w, gcn1_W, gcn1_b, gcn2_W, gcn2_b,
           pool1_W, pool1_b, pool2_W, lx_Wih, lx_Whh, lx_bih, lx_bhh,
           lg_Wih, lg_Whh, lg_bih, lg_bhh):
    tok = x_tokens.astype(_i32)
    wty = word_type.astype(_i32)
    ei = edge_index.astype(_i32)
    row = ei[0]
    col = ei[1]
    ew = edge_attr.astype(_f32)

    embx, deg_p = _sc_prep(tok, wty, col, ew, emb_table, wt_table)
    x, h1, dis2 = _tc_proj(embx, deg_p.reshape(2, _DIS_R, _DIS_C),
                           W_w, b_w.reshape(1, _D), gcn1_W)
    dis = dis2.reshape(_N)

    lxb = (lx_bih + lx_bhh).reshape(1, 4 * _D)
    lgb = (lg_bih + lg_bhh).reshape(1, 4 * _D)

    p = _sc_conv(h1, dis, row, col, ew).reshape(2, _N, _D)
    x1, h2 = _tc_lstm_mid(x, p, gcn1_b.reshape(1, _D),
                          lx_Wih, lx_Whh, lxb, gcn2_W)

    q = _sc_conv(h2, dis, row, col, ew).reshape(2, _N, _D)
    x2, xo = _tc_lstm_fin(x1, q, gcn2_b.reshape(1, _D),
                          lx_Wih, lx_Whh, lxb, mask_pad.reshape(_N, 1))

    g_out = _tc_pool(x1, x2, mask_score.reshape(_N, 1),
                     pool1_W, pool1_b.reshape(1, _D), pool2_W,
                     lg_Wih, lg_Whh, lgb)

    return (xo.reshape(_B, _L, _D), g_out)


# speedup vs baseline: 21.6249x; 1.1164x over previous
import functools

import jax
import jax.numpy as jnp
from jax import lax
from jax.experimental import pallas as pl
from jax.experimental.pallas import tpu as pltpu
from jax.experimental.pallas import tpu_sc as plsc

_N = 10000
_E = 320000
_B = 50
_L = 200
_D = 128

_NC = 2
_NS = 16
_NW = _NC * _NS

_RC = 80
_NCH = _N // _RC
_PE = _E // _NW
_DC = 2000
_EC = 80

_f32 = jnp.float32
_i32 = jnp.int32


def _scale_rows(buf, sbuf, n):
    def body(g, _):
        sv = sbuf[pl.ds(g * 16, 16)]
        for l in range(16):
            i = g * 16 + l
            s = sv[l]
            for j in range(_D // 16):
                sl = pl.ds(j * 16, 16)
                buf[i, sl] = buf[i, sl] * s
        return 0
    lax.fori_loop(0, n // 16, body, 0)


def _zero_rows(buf, n):
    z = jnp.zeros((16,), _f32)
    def body(i, _):
        for j in range(_D // 16):
            buf[i, pl.ds(j * 16, 16)] = z
        return 0
    lax.fori_loop(0, n, body, 0)


def _sc_prep(tok, wty, col, ew, emb_table, wt_table):
    mesh = plsc.VectorSubcoreMesh(core_axis_name="c", subcore_axis_name="s")

    @functools.partial(
        pl.kernel,
        mesh=mesh,
        out_type=(
            jax.ShapeDtypeStruct((_N, _D), _f32),
            jax.ShapeDtypeStruct((2 * _N,), _f32),
        ),
        scratch_types=[
            pltpu.VMEM((_RC,), _i32),
            pltpu.VMEM((_RC,), _i32),
            pltpu.VMEM((_RC, _D), _f32),
            pltpu.VMEM((_RC, _D), _f32),
            pltpu.VMEM((_DC,), _i32),
            pltpu.VMEM((_DC,), _f32),
            pltpu.VMEM_SHARED((_N,), _f32),
            pltpu.SemaphoreType.DMA,
            pltpu.SemaphoreType.DMA,
        ],
    )
    def k(tok_h, wty_h, col_h, ew_h, emb_h, wt_h, embx_o, deg_o,
          tokbuf, wtbuf, ebuf, wbuf, colbuf, ewbuf, deg_acc, sem1, sem2):
        cid = lax.axis_index("c")
        sid = lax.axis_index("s")
        w = sid * _NC + cid

        def zb(i, _):
            ewbuf[pl.ds(i * 16, 16)] = jnp.zeros((16,), _f32)
            return 0
        lax.fori_loop(0, _DC // 16, zb, 0)

        @pl.when(sid < _N // _DC)
        def _():
            pltpu.sync_copy(ewbuf, deg_acc.at[pl.ds(sid * _DC, _DC)])
        plsc.subcore_barrier()

        def dchunk(kk, _):
            base = w * _PE + kk * _DC
            pltpu.sync_copy(col_h.at[pl.ds(base, _DC)], colbuf)
            pltpu.sync_copy(ew_h.at[pl.ds(base, _DC)], ewbuf)
            pltpu.sync_copy(ewbuf, deg_acc.at[colbuf], add=True)
            return 0
        lax.fori_loop(0, _PE // _DC, dchunk, 0)
        plsc.subcore_barrier()

        @pl.when(sid < _N // _DC)
        def _():
            pltpu.sync_copy(deg_acc.at[pl.ds(sid * _DC, _DC)], ewbuf)
            pltpu.sync_copy(ewbuf, deg_o.at[pl.ds(cid * _N + sid * _DC, _DC)])

        for r in range((_NCH + _NW - 1) // _NW):
            ck = w + _NW * r

            @pl.when(ck < _NCH)
            def _():
                base = ck * _RC
                pltpu.sync_copy(tok_h.at[pl.ds(base, _RC)], tokbuf)
                pltpu.sync_copy(wty_h.at[pl.ds(base, _RC)], wtbuf)
                cp1 = pltpu.async_copy(emb_h.at[tokbuf], ebuf, sem1)
                cp2 = pltpu.async_copy(wt_h.at[wtbuf], wbuf, sem2)
                cp1.wait()
                cp2.wait()

                def addb(i, _):
                    for j in range(_D // 16):
                        sl = pl.ds(j * 16, 16)
                        ebuf[i, sl] = ebuf[i, sl] + wbuf[i, sl]
                    return 0
                lax.fori_loop(0, _RC, addb, 0)
                pltpu.sync_copy(ebuf, embx_o.at[pl.ds(base, _RC)])

    return k(tok, wty, col, ew, emb_table, wt_table)


def _sc_conv(h, dis, row, col, ew):
    mesh = plsc.VectorSubcoreMesh(core_axis_name="c", subcore_axis_name="s")

    @functools.partial(
        pl.kernel,
        mesh=mesh,
        out_type=jax.ShapeDtypeStruct((2 * _N, _D), _f32),
        scratch_types=[
            pltpu.VMEM((_PE,), _i32),
            pltpu.VMEM((_EC,), _f32),
            pltpu.VMEM((_EC,), _f32),
            pltpu.VMEM((_EC,), _f32),
            pltpu.VMEM((_EC,), _i32),
            pltpu.VMEM((_EC,), _i32),
            pltpu.VMEM((_EC,), _i32),
            pltpu.VMEM((_EC,), _f32),
            pltpu.VMEM((_EC,), _f32),
            pltpu.VMEM((_EC,), _f32),
            pltpu.VMEM((_EC, _D), _f32),
            pltpu.VMEM((_EC, _D), _f32),
            pltpu.VMEM((_EC, _D), _f32),
            pltpu.VMEM_SHARED((_N, _D), _f32),
            pltpu.SemaphoreType.DMA,
            pltpu.SemaphoreType.DMA,
            pltpu.SemaphoreType.DMA,
            pltpu.SemaphoreType.DMA,
            pltpu.SemaphoreType.DMA,
            pltpu.SemaphoreType.DMA,
            pltpu.SemaphoreType.DMA,
            pltpu.SemaphoreType.DMA,
            pltpu.SemaphoreType.DMA,
        ],
    )
    def k(h_h, dis_h, row_h, col_h, ew_h, out_o,
          rowbig, ewb0, ewb1, ewb2, colb0, colb1, colb2, dgb0, dgb1, dgb2,
          rsb0, rsb1, rsb2, acc,
          gs0, gs1, gs2, as0, as1, as2, ss0, ss1, ss2):
        cid = lax.axis_index("c")
        sid = lax.axis_index("s")
        w = sid * _NC + cid
        ewb = (ewb0, ewb1, ewb2)
        colb = (colb0, colb1, colb2)
        dgb = (dgb0, dgb1, dgb2)
        rsb = (rsb0, rsb1, rsb2)
        gs = (gs0, gs1, gs2)
        aux = (as0, as1, as2)
        ss = (ss0, ss1, ss2)

        @pl.when(cid != 0)
        def _():
            _zero_rows(rsb0, _RC)

        for r in range((_NCH + _NS - 1) // _NS):
            ck = sid + _NS * r

            @pl.when(ck < _NCH)
            def _():
                base = ck * _RC

                @pl.when(cid == 0)
                def _():
                    pltpu.sync_copy(h_h.at[pl.ds(base, _RC)], rsb0)
                    pltpu.sync_copy(dis_h.at[pl.ds(base, _RC)], dgb0)
                    _scale_rows(rsb0, dgb0, _RC)
                pltpu.sync_copy(rsb0, acc.at[pl.ds(base, _RC)])
        plsc.subcore_barrier()

        nch = _PE // _EC

        pltpu.sync_copy(row_h.at[pl.ds(w * _PE, _PE)], rowbig)

        def issue(q, kk):
            idx = rowbig.at[pl.ds(kk * _EC, _EC)]
            base = w * _PE + kk * _EC
            pltpu.async_copy(h_h.at[idx], rsb[q], gs[q])
            pltpu.async_copy(dis_h.at[idx], dgb[q], aux[q])
            pltpu.async_copy(ew_h.at[pl.ds(base, _EC)], ewb[q], aux[q])
            pltpu.async_copy(col_h.at[pl.ds(base, _EC)], colb[q], aux[q])

        def process(q, kk):
            idx = rowbig.at[pl.ds(kk * _EC, _EC)]
            base = w * _PE + kk * _EC
            pltpu.make_async_copy(h_h.at[idx], rsb[q], gs[q]).wait()
            pltpu.make_async_copy(dis_h.at[idx], dgb[q], aux[q]).wait()
            pltpu.make_async_copy(ew_h.at[pl.ds(base, _EC)], ewb[q], aux[q]).wait()
            pltpu.make_async_copy(col_h.at[pl.ds(base, _EC)], colb[q], aux[q]).wait()

            def sb(g, _):
                sv = ewb[q][pl.ds(g * 16, 16)] * dgb[q][pl.ds(g * 16, 16)]
                for l in range(16):
                    i = g * 16 + l
                    s = sv[l]
                    for j in range(_D // 16):
                        sl = pl.ds(j * 16, 16)
                        rsb[q][i, sl] = rsb[q][i, sl] * s
                return 0
            lax.fori_loop(0, _EC // 16, sb, 0)
            pltpu.async_copy(rsb[q], acc.at[colb[q]], ss[q], add=True)

        def wait_scatter(q):
            pltpu.make_async_copy(rsb[q], acc.at[colb[q]], ss[q]).wait()

        issue(0, 0)
        issue(1, 1)
        issue(2, 2)
        process(0, 0)
        wait_scatter(0)
        issue(0, 3)
        process(1, 1)

        def estep(ko, _):
            for b in range(3):
                k = 2 + 3 * ko + b
                q2 = (b + 1) % 3
                wait_scatter(q2)
                issue(q2, k + 2)
                process((b + 2) % 3, k)
            return 0
        lax.fori_loop(0, (nch - 5) // 3, estep, 0)

        kl = nch - 3
        wait_scatter((kl + 2) % 3)
        issue((kl + 2) % 3, kl + 2)
        process(kl % 3, kl)
        process((kl + 1) % 3, kl + 1)
        process((kl + 2) % 3, kl + 2)
        wait_scatter(kl % 3)
        wait_scatter((kl + 1) % 3)
        wait_scatter((kl + 2) % 3)
        plsc.subcore_barrier()

        for r in range((_NCH + _NS - 1) // _NS):
            ck = sid + _NS * r

            @pl.when(ck < _NCH)
            def _():
                base = ck * _RC
                pltpu.sync_copy(acc.at[pl.ds(base, _RC)], rsb0)
                pltpu.sync_copy(dis_h.at[pl.ds(base, _RC)], dgb0)
                _scale_rows(rsb0, dgb0, _RC)
                pltpu.sync_copy(rsb0, out_o.at[pl.ds(cid * _N + base, _RC)])

    return k(h, dis, row, col, ew)


def _dot_t(a, b):
    return lax.dot_general(a, b, (((1,), (1,)), ((), ())),
                           preferred_element_type=_f32)


_R = 1000
_G = _N // _R
_DIS_R, _DIS_C = 80, 125


def _tc_proj(embx, deg3, W_w, b_w, gcn1_W):
    def body(embx_ref, deg_ref, ww_ref, bw_ref, g1w_ref, x_o, h1_o, dis_o):
        xb = _dot_t(embx_ref[...], ww_ref[...]) + bw_ref[...]
        h1 = _dot_t(xb, g1w_ref[...])
        x_o[...] = xb
        h1_o[...] = h1
        deg = deg_ref[0] + deg_ref[1] + 1.0
        dis_o[...] = lax.rsqrt(deg)

    rows_per = _DIS_R // _G
    return pl.pallas_call(
        body,
        grid=(_G,),
        in_specs=[
            pl.BlockSpec((_R, _D), lambda i: (i, 0)),
            pl.BlockSpec((2, rows_per, _DIS_C), lambda i: (0, i, 0)),
            pl.BlockSpec((_D, _D), lambda i: (0, 0)),
            pl.BlockSpec((1, _D), lambda i: (0, 0)),
            pl.BlockSpec((_D, _D), lambda i: (0, 0)),
        ],
        out_specs=[
            pl.BlockSpec((_R, _D), lambda i: (i, 0)),
            pl.BlockSpec((_R, _D), lambda i: (i, 0)),
            pl.BlockSpec((rows_per, _DIS_C), lambda i: (i, 0)),
        ],
        out_shape=[
            jax.ShapeDtypeStruct((_N, _D), _f32),
            jax.ShapeDtypeStruct((_N, _D), _f32),
            jax.ShapeDtypeStruct((_DIS_R, _DIS_C), _f32),
        ],
    )(embx, deg3, W_w, b_w, gcn1_W)


def _lstm2(x0, x1, wih, whh, bl):
    g0 = _dot_t(x0, wih) + bl
    i0 = jax.nn.sigmoid(g0[:, 0:_D])
    gg0 = jnp.tanh(g0[:, 2 * _D:3 * _D])
    o0 = jax.nn.sigmoid(g0[:, 3 * _D:4 * _D])
    c = i0 * gg0
    h = o0 * jnp.tanh(c)
    g1 = _dot_t(x1, wih) + _dot_t(h, whh) + bl
    i1 = jax.nn.sigmoid(g1[:, 0:_D])
    f1 = jax.nn.sigmoid(g1[:, _D:2 * _D])
    gg1 = jnp.tanh(g1[:, 2 * _D:3 * _D])
    o1 = jax.nn.sigmoid(g1[:, 3 * _D:4 * _D])
    c = f1 * c + i1 * gg1
    return o1 * jnp.tanh(c)


def _tc_lstm_mid(x, p, bias_conv, wih, whh, bl, wnext):
    def body(x_ref, p_ref, bc_ref, wih_ref, whh_ref, bl_ref, wn_ref,
             x1_o, h2_o):
        out1 = p_ref[0] + p_ref[1] + bc_ref[...]
        h = _lstm2(x_ref[...], out1, wih_ref[...], whh_ref[...], bl_ref[...])
        x1_o[...] = h
        h2_o[...] = _dot_t(h, wn_ref[...])

    return pl.pallas_call(
        body,
        grid=(_G,),
        in_specs=[
            pl.BlockSpec((_R, _D), lambda i: (i, 0)),
            pl.BlockSpec((2, _R, _D), lambda i: (0, i, 0)),
            pl.BlockSpec((1, _D), lambda i: (0, 0)),
            pl.BlockSpec((4 * _D, _D), lambda i: (0, 0)),
            pl.BlockSpec((4 * _D, _D), lambda i: (0, 0)),
            pl.BlockSpec((1, 4 * _D), lambda i: (0, 0)),
            pl.BlockSpec((_D, _D), lambda i: (0, 0)),
        ],
        out_specs=[
            pl.BlockSpec((_R, _D), lambda i: (i, 0)),
            pl.BlockSpec((_R, _D), lambda i: (i, 0)),
        ],
        out_shape=[
            jax.ShapeDtypeStruct((_N, _D), _f32),
            jax.ShapeDtypeStruct((_N, _D), _f32),
        ],
    )(x, p, bias_conv, wih, whh, bl, wnext)


def _tc_lstm_fin(x, p, bias_conv, wih, whh, bl, mask_col):
    def body(x_ref, p_ref, bc_ref, wih_ref, whh_ref, bl_ref, m_ref,
             x2_o, xo_o):
        out2 = p_ref[0] + p_ref[1] + bc_ref[...]
        h = _lstm2(x_ref[...], out2, wih_ref[...], whh_ref[...], bl_ref[...])
        x2_o[...] = h
        xo_o[...] = jnp.tanh(h) * m_ref[...]

    return pl.pallas_call(
        body,
        grid=(_G,),
        in_specs=[
            pl.BlockSpec((_R, _D), lambda i: (i, 0)),
            pl.BlockSpec((2, _R, _D), lambda i: (0, i, 0)),
            pl.BlockSpec((1, _D), lambda i: (0, 0)),
            pl.BlockSpec((4 * _D, _D), lambda i: (0, 0)),
            pl.BlockSpec((4 * _D, _D), lambda i: (0, 0)),
            pl.BlockSpec((1, 4 * _D), lambda i: (0, 0)),
            pl.BlockSpec((_R, 1), lambda i: (i, 0)),
        ],
        out_specs=[
            pl.BlockSpec((_R, _D), lambda i: (i, 0)),
            pl.BlockSpec((_R, _D), lambda i: (i, 0)),
        ],
        out_shape=[
            jax.ShapeDtypeStruct((_N, _D), _f32),
            jax.ShapeDtypeStruct((_N, _D), _f32),
        ],
    )(x, p, bias_conv, wih, whh, bl, mask_col)


def _tc_pool(x1, x2, ms_col, p1w, p1b, p2w, wih, whh, bl):
    def pool_one(xf, ms, p1w_v, p1b_v, p2w_v):
        hh = jnp.tanh(_dot_t(xf, p1w_v) + p1b_v)
        s = jnp.sum(hh * p2w_v, axis=1, keepdims=True) + ms
        s3 = s.reshape(_B, _L, 1)
        m = jnp.max(s3, axis=1, keepdims=True)
        e = jnp.exp(s3 - m)
        den = jnp.sum(e, axis=1, keepdims=True)
        alpha = e / den
        x3 = xf.reshape(_B, _L, _D)
        return jnp.sum(alpha * x3, axis=1)

    def body(x1_ref, x2_ref, ms_ref, p1w_ref, p1b_ref, p2w_ref,
             wih_ref, whh_ref, bl_ref, go_o):
        ms = ms_ref[...]
        g1 = pool_one(x1_ref[...], ms, p1w_ref[...], p1b_ref[...], p2w_ref[...])
        g2 = pool_one(x2_ref[...], ms, p1w_ref[...], p1b_ref[...], p2w_ref[...])
        go_o[...] = _lstm2(g1, g2, wih_ref[...], whh_ref[...], bl_ref[...])

    return pl.pallas_call(
        body,
        grid=(1,),
        in_specs=[
            pl.BlockSpec((_N, _D), lambda i: (0, 0)),
            pl.BlockSpec((_N, _D), lambda i: (0, 0)),
            pl.BlockSpec((_N, 1), lambda i: (0, 0)),
            pl.BlockSpec((_D, _D), lambda i: (0, 0)),
            pl.BlockSpec((1, _D), lambda i: (0, 0)),
            pl.BlockSpec((1, _D), lambda i: (0, 0)),
            pl.BlockSpec((4 * _D, _D), lambda i: (0, 0)),
            pl.BlockSpec((4 * _D, _D), lambda i: (0, 0)),
            pl.BlockSpec((1, 4 * _D), lambda i: (0, 0)),
        ],
        out_specs=pl.BlockSpec((_B, _D), lambda i: (0, 0)),
        out_shape=jax.ShapeDtypeStruct((_B, _D), _f32),
    )(x1, x2, ms_col, p1w, p1b, p2w, wih, whh, bl)


def kernel(x_tokens, word_type, edge_index, edge_attr, mask_pad, mask_score,
           emb_table, wt_table, W_w, b_w, gcn1_W, gcn1_b, gcn2_W, gcn2_b,
           pool1_W, pool1_b, pool2_W, lx_Wih, lx_Whh, lx_bih, lx_bhh,
           lg_Wih, lg_Whh, lg_bih, lg_bhh):
    tok = x_tokens.astype(_i32)
    wty = word_type.astype(_i32)
    ei = edge_index.astype(_i32)
    row = ei[0]
    col = ei[1]
    ew = edge_attr.astype(_f32)

    embx, deg_p = _sc_prep(tok, wty, col, ew, emb_table, wt_table)
    x, h1, dis2 = _tc_proj(embx, deg_p.reshape(2, _DIS_R, _DIS_C),
                           W_w, b_w.reshape(1, _D), gcn1_W)
    dis = dis2.reshape(_N)

    lxb = (lx_bih + lx_bhh).reshape(1, 4 * _D)
    lgb = (lg_bih + lg_bhh).reshape(1, 4 * _D)

    p = _sc_conv(h1, dis, row, col, ew).reshape(2, _N, _D)
    x1, h2 = _tc_lstm_mid(x, p, gcn1_b.reshape(1, _D),
                          lx_Wih, lx_Whh, lxb, gcn2_W)

    q = _sc_conv(h2, dis, row, col, ew).reshape(2, _N, _D)
    x2, xo = _tc_lstm_fin(x1, q, gcn2_b.reshape(1, _D),
                          lx_Wih, lx_Whh, lxb, mask_pad.reshape(_N, 1))

    g_out = _tc_pool(x1, x2, mask_score.reshape(_N, 1),
                     pool1_W, pool1_b.reshape(1, _D), pool2_W,
                     lg_Wih, lg_Whh, lgb)

    return (xo.reshape(_B, _L, _D), g_out)


# speedup vs baseline: 24.2658x; 1.1221x over previous
import functools

import jax
import jax.numpy as jnp
from jax import lax
from jax.experimental import pallas as pl
from jax.experimental.pallas import tpu as pltpu
from jax.experimental.pallas import tpu_sc as plsc

_N = 10000
_E = 320000
_B = 50
_L = 200
_D = 128

_NC = 2
_NS = 16
_NW = _NC * _NS

_RC = 80
_NCH = _N // _RC
_PE = _E // _NW
_DC = 2000
_EC = 80

_f32 = jnp.float32
_i32 = jnp.int32


def _scale_rows(buf, sbuf, n):
    def body(g, _):
        sv = sbuf[pl.ds(g * 16, 16)]
        for l in range(16):
            i = g * 16 + l
            s = sv[l]
            for j in range(_D // 16):
                sl = pl.ds(j * 16, 16)
                buf[i, sl] = buf[i, sl] * s
        return 0
    lax.fori_loop(0, n // 16, body, 0)


def _zero_rows(buf, n):
    z = jnp.zeros((16,), _f32)
    def body(i, _):
        for j in range(_D // 16):
            buf[i, pl.ds(j * 16, 16)] = z
        return 0
    lax.fori_loop(0, n, body, 0)


def _sc_prep(tok, col, ew, emb_table):
    mesh = plsc.VectorSubcoreMesh(core_axis_name="c", subcore_axis_name="s")
    nebc = (_NCH + _NW - 1) // _NW

    @functools.partial(
        pl.kernel,
        mesh=mesh,
        out_type=(
            jax.ShapeDtypeStruct((_N, _D), _f32),
            jax.ShapeDtypeStruct((2 * _N,), _f32),
        ),
        scratch_types=[
            pltpu.VMEM((nebc * _RC,), _i32),
            pltpu.VMEM((_RC, _D), _f32),
            pltpu.VMEM((_RC, _D), _f32),
            pltpu.VMEM((_RC, _D), _f32),
            pltpu.VMEM((_RC, _D), _f32),
            pltpu.VMEM((_DC,), _i32),
            pltpu.VMEM((_DC,), _i32),
            pltpu.VMEM((_DC,), _f32),
            pltpu.VMEM((_DC,), _f32),
            pltpu.VMEM_SHARED((_N,), _f32),
            pltpu.SemaphoreType.DMA,
            pltpu.SemaphoreType.DMA,
            pltpu.SemaphoreType.DMA,
            pltpu.SemaphoreType.DMA,
            pltpu.SemaphoreType.DMA,
            pltpu.SemaphoreType.DMA,
            pltpu.SemaphoreType.DMA,
            pltpu.SemaphoreType.DMA,
        ],
    )
    def k(tok_h, col_h, ew_h, emb_h, embx_o, deg_o,
          tokb, eb0, eb1, eb2, eb3, colb0, colb1, ewb0, ewb1, deg_acc,
          gr0, gr1, gr2, gr3, cs0, cs1, ds0, ds1):
        cid = lax.axis_index("c")
        sid = lax.axis_index("s")
        w = sid * _NC + cid
        eb = (eb0, eb1, eb2, eb3)
        gr = (gr0, gr1, gr2, gr3)
        colb = (colb0, colb1)
        ewb = (ewb0, ewb1)
        cs = (cs0, cs1)
        ds = (ds0, ds1)

        def zb(i, _):
            ewb0[pl.ds(i * 16, 16)] = jnp.zeros((16,), _f32)
            return 0
        lax.fori_loop(0, _DC // 16, zb, 0)

        @pl.when(sid < _N // _DC)
        def _():
            pltpu.sync_copy(ewb0, deg_acc.at[pl.ds(sid * _DC, _DC)])
        plsc.subcore_barrier()

        for r in range(nebc):
            ck = w + _NW * r

            @pl.when(ck < _NCH)
            def _():
                pltpu.sync_copy(tok_h.at[pl.ds(ck * _RC, _RC)],
                                tokb.at[pl.ds(r * _RC, _RC)])
                pltpu.async_copy(emb_h.at[tokb.at[pl.ds(r * _RC, _RC)]],
                                 eb[r], gr[r])

        def dload(q, kk):
            base = w * _PE + kk * _DC
            pltpu.async_copy(col_h.at[pl.ds(base, _DC)], colb[q], cs[q])
            pltpu.async_copy(ew_h.at[pl.ds(base, _DC)], ewb[q], cs[q])

        def dproc(q, kk):
            base = w * _PE + kk * _DC
            pltpu.make_async_copy(col_h.at[pl.ds(base, _DC)], colb[q], cs[q]).wait()
            pltpu.make_async_copy(ew_h.at[pl.ds(base, _DC)], ewb[q], cs[q]).wait()
            pltpu.async_copy(ewb[q], deg_acc.at[colb[q]], ds[q], add=True)

        def dwaitscat(q):
            pltpu.make_async_copy(ewb[q], deg_acc.at[colb[q]], ds[q]).wait()

        dload(0, 0)
        dload(1, 1)
        dproc(0, 0)
        dproc(1, 1)
        dwaitscat(0)
        dload(0, 2)
        dproc(0, 2)
        dwaitscat(1)
        dload(1, 3)
        dproc(1, 3)
        dwaitscat(0)
        dload(0, 4)
        dproc(0, 4)
        dwaitscat(1)
        dwaitscat(0)

        for r in range(nebc):
            ck = w + _NW * r

            @pl.when(ck < _NCH)
            def _():
                pltpu.make_async_copy(emb_h.at[tokb.at[pl.ds(r * _RC, _RC)]],
                                      eb[r], gr[r]).wait()
                pltpu.sync_copy(eb[r], embx_o.at[pl.ds(ck * _RC, _RC)])

        plsc.subcore_barrier()

        @pl.when(sid < _N // _DC)
        def _():
            pltpu.sync_copy(deg_acc.at[pl.ds(sid * _DC, _DC)], ewb0)
            pltpu.sync_copy(ewb0, deg_o.at[pl.ds(cid * _N + sid * _DC, _DC)])

    return k(tok, col, ew, emb_table)


def _sc_conv(h, dis, row, col, ew):
    mesh = plsc.VectorSubcoreMesh(core_axis_name="c", subcore_axis_name="s")

    @functools.partial(
        pl.kernel,
        mesh=mesh,
        out_type=jax.ShapeDtypeStruct((2 * _N, _D), _f32),
        scratch_types=[
            pltpu.VMEM((_PE,), _i32),
            pltpu.VMEM((_EC,), _f32),
            pltpu.VMEM((_EC,), _f32),
            pltpu.VMEM((_EC,), _f32),
            pltpu.VMEM((_EC,), _i32),
            pltpu.VMEM((_EC,), _i32),
            pltpu.VMEM((_EC,), _i32),
            pltpu.VMEM((_EC,), _f32),
            pltpu.VMEM((_EC,), _f32),
            pltpu.VMEM((_EC,), _f32),
            pltpu.VMEM((_EC, _D), _f32),
            pltpu.VMEM((_EC, _D), _f32),
            pltpu.VMEM((_EC, _D), _f32),
            pltpu.VMEM_SHARED((_N, _D), _f32),
            pltpu.SemaphoreType.DMA,
            pltpu.SemaphoreType.DMA,
            pltpu.SemaphoreType.DMA,
            pltpu.SemaphoreType.DMA,
            pltpu.SemaphoreType.DMA,
            pltpu.SemaphoreType.DMA,
            pltpu.SemaphoreType.DMA,
            pltpu.SemaphoreType.DMA,
            pltpu.SemaphoreType.DMA,
        ],
    )
    def k(h_h, dis_h, row_h, col_h, ew_h, out_o,
          rowbig, ewb0, ewb1, ewb2, colb0, colb1, colb2, dgb0, dgb1, dgb2,
          rsb0, rsb1, rsb2, acc,
          gs0, gs1, gs2, as0, as1, as2, ss0, ss1, ss2):
        cid = lax.axis_index("c")
        sid = lax.axis_index("s")
        w = sid * _NC + cid
        ewb = (ewb0, ewb1, ewb2)
        colb = (colb0, colb1, colb2)
        dgb = (dgb0, dgb1, dgb2)
        rsb = (rsb0, rsb1, rsb2)
        gs = (gs0, gs1, gs2)
        aux = (as0, as1, as2)
        ss = (ss0, ss1, ss2)

        @pl.when(cid != 0)
        def _():
            _zero_rows(rsb0, _RC)

        for r in range((_NCH + _NS - 1) // _NS):
            ck = sid + _NS * r

            @pl.when(ck < _NCH)
            def _():
                base = ck * _RC

                @pl.when(cid == 0)
                def _():
                    pltpu.sync_copy(h_h.at[pl.ds(base, _RC)], rsb0)
                    pltpu.sync_copy(dis_h.at[pl.ds(base, _RC)], dgb0)
                    _scale_rows(rsb0, dgb0, _RC)
                pltpu.sync_copy(rsb0, acc.at[pl.ds(base, _RC)])
        plsc.subcore_barrier()

        nch = _PE // _EC

        pltpu.sync_copy(row_h.at[pl.ds(w * _PE, _PE)], rowbig)

        def issue(q, kk):
            idx = rowbig.at[pl.ds(kk * _EC, _EC)]
            base = w * _PE + kk * _EC
            pltpu.async_copy(h_h.at[idx], rsb[q], gs[q])
            pltpu.async_copy(dis_h.at[idx], dgb[q], aux[q])
            pltpu.async_copy(ew_h.at[pl.ds(base, _EC)], ewb[q], aux[q])
            pltpu.async_copy(col_h.at[pl.ds(base, _EC)], colb[q], aux[q])

        def process(q, kk):
            idx = rowbig.at[pl.ds(kk * _EC, _EC)]
            base = w * _PE + kk * _EC
            pltpu.make_async_copy(h_h.at[idx], rsb[q], gs[q]).wait()
            pltpu.make_async_copy(dis_h.at[idx], dgb[q], aux[q]).wait()
            pltpu.make_async_copy(ew_h.at[pl.ds(base, _EC)], ewb[q], aux[q]).wait()
            pltpu.make_async_copy(col_h.at[pl.ds(base, _EC)], colb[q], aux[q]).wait()

            def sb(g, _):
                sv = ewb[q][pl.ds(g * 16, 16)] * dgb[q][pl.ds(g * 16, 16)]
                for l in range(16):
                    i = g * 16 + l
                    s = sv[l]
                    for j in range(_D // 16):
                        sl = pl.ds(j * 16, 16)
                        rsb[q][i, sl] = rsb[q][i, sl] * s
                return 0
            lax.fori_loop(0, _EC // 16, sb, 0)
            pltpu.async_copy(rsb[q], acc.at[colb[q]], ss[q], add=True)

        def wait_scatter(q):
            pltpu.make_async_copy(rsb[q], acc.at[colb[q]], ss[q]).wait()

        issue(0, 0)
        issue(1, 1)
        issue(2, 2)
        process(0, 0)
        wait_scatter(0)
        issue(0, 3)
        process(1, 1)

        def estep(ko, _):
            for b in range(3):
                k = 2 + 3 * ko + b
                q2 = (b + 1) % 3
                wait_scatter(q2)
                issue(q2, k + 2)
                process((b + 2) % 3, k)
            return 0
        lax.fori_loop(0, (nch - 5) // 3, estep, 0)

        kl = nch - 3
        wait_scatter((kl + 2) % 3)
        issue((kl + 2) % 3, kl + 2)
        process(kl % 3, kl)
        process((kl + 1) % 3, kl + 1)
        process((kl + 2) % 3, kl + 2)
        wait_scatter(kl % 3)
        wait_scatter((kl + 1) % 3)
        wait_scatter((kl + 2) % 3)
        plsc.subcore_barrier()

        for r in range((_NCH + _NS - 1) // _NS):
            ck = sid + _NS * r

            @pl.when(ck < _NCH)
            def _():
                base = ck * _RC
                pltpu.sync_copy(acc.at[pl.ds(base, _RC)], rsb0)
                pltpu.sync_copy(dis_h.at[pl.ds(base, _RC)], dgb0)
                _scale_rows(rsb0, dgb0, _RC)
                pltpu.sync_copy(rsb0, out_o.at[pl.ds(cid * _N + base, _RC)])

    return k(h, dis, row, col, ew)


def _dot_t(a, b):
    return lax.dot_general(a, b, (((1,), (1,)), ((), ())),
                           preferred_element_type=_f32)


_R = 1000
_G = _N // _R
_DIS_R, _DIS_C = 80, 125


def _tc_proj(embx, wty_col, wt_table, deg3, W_w, b_w, gcn1_W):
    def body(embx_ref, wty_ref, wt_ref, deg_ref, ww_ref, bw_ref, g1w_ref,
             x_o, h1_o, dis_o):
        ntypes = wt_ref.shape[0]
        iot = lax.broadcasted_iota(jnp.int32, (1, ntypes), 1).astype(_f32)
        oh = (wty_ref[...] == iot)
        wtv = lax.dot_general(oh.astype(_f32), wt_ref[...],
                              (((1,), (0,)), ((), ())),
                              preferred_element_type=_f32)
        xb = _dot_t(embx_ref[...] + wtv, ww_ref[...]) + bw_ref[...]
        h1 = _dot_t(xb, g1w_ref[...])
        x_o[...] = xb
        h1_o[...] = h1
        deg = deg_ref[0] + deg_ref[1] + 1.0
        dis_o[...] = lax.rsqrt(deg)

    rows_per = _DIS_R // _G
    nt = wt_table.shape[0]
    return pl.pallas_call(
        body,
        grid=(_G,),
        in_specs=[
            pl.BlockSpec((_R, _D), lambda i: (i, 0)),
            pl.BlockSpec((_R, 1), lambda i: (i, 0)),
            pl.BlockSpec((nt, _D), lambda i: (0, 0)),
            pl.BlockSpec((2, rows_per, _DIS_C), lambda i: (0, i, 0)),
            pl.BlockSpec((_D, _D), lambda i: (0, 0)),
            pl.BlockSpec((1, _D), lambda i: (0, 0)),
            pl.BlockSpec((_D, _D), lambda i: (0, 0)),
        ],
        out_specs=[
            pl.BlockSpec((_R, _D), lambda i: (i, 0)),
            pl.BlockSpec((_R, _D), lambda i: (i, 0)),
            pl.BlockSpec((rows_per, _DIS_C), lambda i: (i, 0)),
        ],
        out_shape=[
            jax.ShapeDtypeStruct((_N, _D), _f32),
            jax.ShapeDtypeStruct((_N, _D), _f32),
            jax.ShapeDtypeStruct((_DIS_R, _DIS_C), _f32),
        ],
    )(embx, wty_col, wt_table, deg3, W_w, b_w, gcn1_W)


def _lstm2(x0, x1, wih, whh, bl):
    g0 = _dot_t(x0, wih) + bl
    i0 = jax.nn.sigmoid(g0[:, 0:_D])
    gg0 = jnp.tanh(g0[:, 2 * _D:3 * _D])
    o0 = jax.nn.sigmoid(g0[:, 3 * _D:4 * _D])
    c = i0 * gg0
    h = o0 * jnp.tanh(c)
    g1 = _dot_t(x1, wih) + _dot_t(h, whh) + bl
    i1 = jax.nn.sigmoid(g1[:, 0:_D])
    f1 = jax.nn.sigmoid(g1[:, _D:2 * _D])
    gg1 = jnp.tanh(g1[:, 2 * _D:3 * _D])
    o1 = jax.nn.sigmoid(g1[:, 3 * _D:4 * _D])
    c = f1 * c + i1 * gg1
    return o1 * jnp.tanh(c)


def _tc_lstm_mid(x, p, bias_conv, wih, whh, bl, wnext):
    def body(x_ref, p_ref, bc_ref, wih_ref, whh_ref, bl_ref, wn_ref,
             x1_o, h2_o):
        out1 = p_ref[0] + p_ref[1] + bc_ref[...]
        h = _lstm2(x_ref[...], out1, wih_ref[...], whh_ref[...], bl_ref[...])
        x1_o[...] = h
        h2_o[...] = _dot_t(h, wn_ref[...])

    return pl.pallas_call(
        body,
        grid=(_G,),
        in_specs=[
            pl.BlockSpec((_R, _D), lambda i: (i, 0)),
            pl.BlockSpec((2, _R, _D), lambda i: (0, i, 0)),
            pl.BlockSpec((1, _D), lambda i: (0, 0)),
            pl.BlockSpec((4 * _D, _D), lambda i: (0, 0)),
            pl.BlockSpec((4 * _D, _D), lambda i: (0, 0)),
            pl.BlockSpec((1, 4 * _D), lambda i: (0, 0)),
            pl.BlockSpec((_D, _D), lambda i: (0, 0)),
        ],
        out_specs=[
            pl.BlockSpec((_R, _D), lambda i: (i, 0)),
            pl.BlockSpec((_R, _D), lambda i: (i, 0)),
        ],
        out_shape=[
            jax.ShapeDtypeStruct((_N, _D), _f32),
            jax.ShapeDtypeStruct((_N, _D), _f32),
        ],
    )(x, p, bias_conv, wih, whh, bl, wnext)


def _tc_lstm_fin(x, p, bias_conv, wih, whh, bl, mask_col):
    def body(x_ref, p_ref, bc_ref, wih_ref, whh_ref, bl_ref, m_ref,
             x2_o, xo_o):
        out2 = p_ref[0] + p_ref[1] + bc_ref[...]
        h = _lstm2(x_ref[...], out2, wih_ref[...], whh_ref[...], bl_ref[...])
        x2_o[...] = h
        xo_o[...] = jnp.tanh(h) * m_ref[...]

    return pl.pallas_call(
        body,
        grid=(_G,),
        in_specs=[
            pl.BlockSpec((_R, _D), lambda i: (i, 0)),
            pl.BlockSpec((2, _R, _D), lambda i: (0, i, 0)),
            pl.BlockSpec((1, _D), lambda i: (0, 0)),
            pl.BlockSpec((4 * _D, _D), lambda i: (0, 0)),
            pl.BlockSpec((4 * _D, _D), lambda i: (0, 0)),
            pl.BlockSpec((1, 4 * _D), lambda i: (0, 0)),
            pl.BlockSpec((_R, 1), lambda i: (i, 0)),
        ],
        out_specs=[
            pl.BlockSpec((_R, _D), lambda i: (i, 0)),
            pl.BlockSpec((_R, _D), lambda i: (i, 0)),
        ],
        out_shape=[
            jax.ShapeDtypeStruct((_N, _D), _f32),
            jax.ShapeDtypeStruct((_N, _D), _f32),
        ],
    )(x, p, bias_conv, wih, whh, bl, mask_col)


def _tc_pool(x1, x2, ms_col, p1w, p1b, p2w, wih, whh, bl):
    def pool_one(xf, ms, p1w_v, p1b_v, p2w_v):
        hh = jnp.tanh(_dot_t(xf, p1w_v) + p1b_v)
        s = jnp.sum(hh * p2w_v, axis=1, keepdims=True) + ms
        s3 = s.reshape(_B, _L, 1)
        m = jnp.max(s3, axis=1, keepdims=True)
        e = jnp.exp(s3 - m)
        den = jnp.sum(e, axis=1, keepdims=True)
        alpha = e / den
        x3 = xf.reshape(_B, _L, _D)
        return jnp.sum(alpha * x3, axis=1)

    def body(x1_ref, x2_ref, ms_ref, p1w_ref, p1b_ref, p2w_ref,
             wih_ref, whh_ref, bl_ref, go_o):
        ms = ms_ref[...]
        g1 = pool_one(x1_ref[...], ms, p1w_ref[...], p1b_ref[...], p2w_ref[...])
        g2 = pool_one(x2_ref[...], ms, p1w_ref[...], p1b_ref[...], p2w_ref[...])
        go_o[...] = _lstm2(g1, g2, wih_ref[...], whh_ref[...], bl_ref[...])

    return pl.pallas_call(
        body,
        grid=(1,),
        in_specs=[
            pl.BlockSpec((_N, _D), lambda i: (0, 0)),
            pl.BlockSpec((_N, _D), lambda i: (0, 0)),
            pl.BlockSpec((_N, 1), lambda i: (0, 0)),
            pl.BlockSpec((_D, _D), lambda i: (0, 0)),
            pl.BlockSpec((1, _D), lambda i: (0, 0)),
            pl.BlockSpec((1, _D), lambda i: (0, 0)),
            pl.BlockSpec((4 * _D, _D), lambda i: (0, 0)),
            pl.BlockSpec((4 * _D, _D), lambda i: (0, 0)),
            pl.BlockSpec((1, 4 * _D), lambda i: (0, 0)),
        ],
        out_specs=pl.BlockSpec((_B, _D), lambda i: (0, 0)),
        out_shape=jax.ShapeDtypeStruct((_B, _D), _f32),
    )(x1, x2, ms_col, p1w, p1b, p2w, wih, whh, bl)


def kernel(x_tokens, word_type, edge_index, edge_attr, mask_pad, mask_score,
           emb_table, wt_table, W_w, b_w, gcn1_W, gcn1_b, gcn2_W, gcn2_b,
           pool1_W, pool1_b, pool2_W, lx_Wih, lx_Whh, lx_bih, lx_bhh,
           lg_Wih, lg_Whh, lg_bih, lg_bhh):
    tok = x_tokens.astype(_i32)
    wty = word_type.astype(_i32)
    ei = edge_index.astype(_i32)
    row = ei[0]
    col = ei[1]
    ew = edge_attr.astype(_f32)

    embx, deg_p = _sc_prep(tok, col, ew, emb_table)
    wty_col = wty.astype(_f32).reshape(_N, 1)
    x, h1, dis2 = _tc_proj(embx, wty_col, wt_table,
                           deg_p.reshape(2, _DIS_R, _DIS_C),
                           W_w, b_w.reshape(1, _D), gcn1_W)
    dis = dis2.reshape(_N)

    lxb = (lx_bih + lx_bhh).reshape(1, 4 * _D)
    lgb = (lg_bih + lg_bhh).reshape(1, 4 * _D)

    p = _sc_conv(h1, dis, row, col, ew).reshape(2, _N, _D)
    x1, h2 = _tc_lstm_mid(x, p, gcn1_b.reshape(1, _D),
                          lx_Wih, lx_Whh, lxb, gcn2_W)

    q = _sc_conv(h2, dis, row, col, ew).reshape(2, _N, _D)
    x2, xo = _tc_lstm_fin(x1, q, gcn2_b.reshape(1, _D),
                          lx_Wih, lx_Whh, lxb, mask_pad.reshape(_N, 1))

    g_out = _tc_pool(x1, x2, mask_score.reshape(_N, 1),
                     pool1_W, pool1_b.reshape(1, _D), pool2_W,
                     lg_Wih, lg_Whh, lgb)

    return (xo.reshape(_B, _L, _D), g_out)
